# baseline (speedup 1.0000x reference)
import jax
import jax.numpy as jnp
from jax import lax
from jax.experimental import pallas as pl
from jax.experimental.pallas import tpu as pltpu


def kernel(x, pos_table):
    B, T, E = x.shape
    N = B * T
    RB = 1024
    NBUF = 6
    nblk = N // RB
    blk_per_batch = T // RB
    x2 = x.reshape(N, E)

    def _kern(x_hbm, pt_hbm, o_hbm, pos_v, inb, outb, psem, isems, osems):
        cp = pltpu.make_async_copy(pt_hbm.at[pl.ds(0, B)], pos_v, psem)
        cp.start()
        cp.wait()

        def in_copy(g, sl):
            return pltpu.make_async_copy(
                x_hbm.at[pl.ds(g * RB, RB)], inb.at[sl], isems.at[sl])

        def out_copy(g, sl):
            return pltpu.make_async_copy(
                outb.at[sl], o_hbm.at[pl.ds(g * RB, RB)], osems.at[sl])

        for g in range(NBUF):
            in_copy(g, g).start()

        def loop_body(g, carry):
            sl = lax.rem(g, NBUF)
            in_copy(g, sl).wait()

            @pl.when(g >= NBUF)
            def _():
                out_copy(g - NBUF, sl).wait()

            b = g // blk_per_batch
            outb[sl] = inb[sl] + pos_v[pl.ds(b, 1), :]
            out_copy(g, sl).start()

            @pl.when(g + NBUF < nblk)
            def _():
                in_copy(g + NBUF, sl).start()

            return carry

        lax.fori_loop(0, nblk, loop_body, 0)
        for i in range(NBUF):
            g = nblk - NBUF + i
            out_copy(g, g % NBUF).wait()

    out = pl.pallas_call(
        _kern,
        grid=(),
        in_specs=[
            pl.BlockSpec(memory_space=pl.ANY),
            pl.BlockSpec(memory_space=pl.ANY),
        ],
        out_specs=pl.BlockSpec(memory_space=pl.ANY),
        out_shape=jax.ShapeDtypeStruct((N, E), x.dtype),
        scratch_shapes=[
            pltpu.VMEM((B, E), jnp.float32),
            pltpu.VMEM((NBUF, RB, E), jnp.float32),
            pltpu.VMEM((NBUF, RB, E), jnp.float32),
            pltpu.SemaphoreType.DMA,
            pltpu.SemaphoreType.DMA((NBUF,)),
            pltpu.SemaphoreType.DMA((NBUF,)),
        ],
    )(x2, pos_table)
    return out.reshape(B, T, E)


# speedup vs baseline: 1.0023x; 1.0023x over previous
import jax
import jax.numpy as jnp
from jax import lax
from jax.experimental import pallas as pl
from jax.experimental.pallas import tpu as pltpu


def kernel(x, pos_table):
    B, T, E = x.shape
    N = B * T
    RB = 1024
    NBUF = 4
    nblk = N // RB
    blk_per_batch = T // RB
    x2 = x.reshape(N, E)

    def _kern(x_hbm, pt_hbm, o_hbm, pos_v, inb, outb, psem, isems, osems):
        cp = pltpu.make_async_copy(pt_hbm.at[pl.ds(0, B)], pos_v, psem)
        cp.start()
        cp.wait()

        def in_copy(g, sl):
            return pltpu.make_async_copy(
                x_hbm.at[pl.ds(g * RB, RB)], inb.at[sl], isems.at[sl])

        def out_copy(g, sl):
            return pltpu.make_async_copy(
                outb.at[sl], o_hbm.at[pl.ds(g * RB, RB)], osems.at[sl])

        for g in range(NBUF):
            in_copy(g, g).start()

        def loop_body(g, carry):
            sl = lax.rem(g, NBUF)
            in_copy(g, sl).wait()

            @pl.when(g >= NBUF)
            def _():
                out_copy(g - NBUF, sl).wait()

            b = g // blk_per_batch
            outb[sl] = inb[sl] + pos_v[pl.ds(b, 1), :]
            out_copy(g, sl).start()

            @pl.when(g + NBUF < nblk)
            def _():
                in_copy(g + NBUF, sl).start()

            return carry

        lax.fori_loop(0, nblk, loop_body, 0)
        for i in range(NBUF):
            g = nblk - NBUF + i
            out_copy(g, g % NBUF).wait()

    out = pl.pallas_call(
        _kern,
        grid=(),
        in_specs=[
            pl.BlockSpec(memory_space=pl.ANY),
            pl.BlockSpec(memory_space=pl.ANY),
        ],
        out_specs=pl.BlockSpec(memory_space=pl.ANY),
        out_shape=jax.ShapeDtypeStruct((N, E), x.dtype),
        scratch_shapes=[
            pltpu.VMEM((B, E), jnp.float32),
            pltpu.VMEM((NBUF, RB, E), jnp.float32),
            pltpu.VMEM((NBUF, RB, E), jnp.float32),
            pltpu.SemaphoreType.DMA,
            pltpu.SemaphoreType.DMA((NBUF,)),
            pltpu.SemaphoreType.DMA((NBUF,)),
        ],
    )(x2, pos_table)
    return out.reshape(B, T, E)
